# Initial kernel scaffold; baseline (speedup 1.0000x reference)
import jax
import jax.numpy as jnp
from jax.experimental import pallas as pl


def kernel(head_emb, tail_emb, rel_ids, W, b):
    raise NotImplementedError("write your pallas kernel here")


# speedup vs baseline: 3.5867x; 3.5867x over previous
import functools

import jax
import jax.numpy as jnp
from jax import lax
from jax.experimental import pallas as pl
from jax.experimental.pallas import tpu as pltpu
from jax.experimental.pallas import tpu_sc as plsc

B = 16384
D = 128
NUM_REL = 1000
NREL_PAD = 1024
NC = 2
NS = 16
NW = NC * NS
RPW = B // NW
SB = 128
NSB = RPW // SB
LANES = 16
NG = SB // LANES
KCH = D // LANES


def _sc_body(h_hbm, t_hbm, idx_hbm, w_hbm, b_hbm, out_hbm,
             idx_v, b_v, out_v, sc_v,
             h0, t0, w0, h1, t1, w1, sem0, sem1):
    wid = lax.axis_index("s") * NC + lax.axis_index("c")
    base = wid * RPW

    pltpu.sync_copy(idx_hbm.at[pl.ds(base, RPW)], idx_v)
    pltpu.sync_copy(b_hbm, b_v)

    bufs = ((h0, t0, w0, sem0), (h1, t1, w1, sem1))

    def start(sb_i):
        hb, tb, wb, sem = bufs[sb_i % 2]
        off = sb_i * SB
        c1 = pltpu.async_copy(h_hbm.at[pl.ds(base + off, SB)], hb, sem)
        c2 = pltpu.async_copy(t_hbm.at[pl.ds(base + off, SB)], tb, sem)
        c3 = pltpu.async_copy(w_hbm.at[idx_v.at[pl.ds(off, SB)]], wb, sem)
        return (c1, c2, c3)

    iota = lax.iota(jnp.int32, LANES)
    inflight = start(0)
    for sb_i in range(NSB):
        hb, tb, wb, _ = bufs[sb_i % 2]
        cur = inflight
        if sb_i + 1 < NSB:
            inflight = start(sb_i + 1)
        for c in cur:
            c.wait()

        def group(g, carry, sb_i=sb_i, hb=hb, tb=tb, wb=wb):
            goff = sb_i * SB + g * LANES
            idx16 = idx_v[pl.ds(goff, LANES)]
            tot = plsc.load_gather(b_v, [idx16])
            row0 = g * LANES
            for r in range(LANES):
                row = row0 + r
                s0 = pl.ds(0, LANES)
                acc = hb[row, s0] * tb[row, s0] * wb[row, s0]
                for k in range(1, KCH):
                    s = pl.ds(k * LANES, LANES)
                    acc = acc + hb[row, s] * tb[row, s] * wb[row, s]
                sc_v[r] = acc
            for col in range(LANES):
                cidx = jnp.full((LANES,), col, jnp.int32)
                tot = tot + plsc.load_gather(sc_v, [iota, cidx])
            out_v[pl.ds(goff, LANES)] = tot
            return carry

        lax.fori_loop(0, NG, group, None)

    pltpu.sync_copy(out_v, out_hbm.at[pl.ds(base, RPW)])


@jax.jit
def _link_predict(head_emb, tail_emb, rel_ids, W, b_pad):
    mesh = plsc.VectorSubcoreMesh(core_axis_name="c", subcore_axis_name="s")
    f = pl.kernel(
        _sc_body,
        out_type=jax.ShapeDtypeStruct((B,), jnp.float32),
        mesh=mesh,
        compiler_params=pltpu.CompilerParams(needs_layout_passes=False),
        scratch_types=[
            pltpu.VMEM((RPW,), jnp.int32),
            pltpu.VMEM((NREL_PAD,), jnp.float32),
            pltpu.VMEM((RPW,), jnp.float32),
            pltpu.VMEM((LANES, LANES), jnp.float32),
            pltpu.VMEM((SB, D), jnp.float32),
            pltpu.VMEM((SB, D), jnp.float32),
            pltpu.VMEM((SB, D), jnp.float32),
            pltpu.VMEM((SB, D), jnp.float32),
            pltpu.VMEM((SB, D), jnp.float32),
            pltpu.VMEM((SB, D), jnp.float32),
            pltpu.SemaphoreType.DMA,
            pltpu.SemaphoreType.DMA,
        ],
    )
    return f(head_emb, tail_emb, rel_ids, W, b_pad)


def kernel(head_emb, tail_emb, rel_ids, W, b):
    idx = rel_ids.astype(jnp.int32)
    b_pad = jnp.zeros((NREL_PAD,), jnp.float32).at[:NUM_REL].set(b)
    return _link_predict(head_emb, tail_emb, idx, W, b_pad)
